# Initial kernel scaffold; baseline (speedup 1.0000x reference)
import jax
import jax.numpy as jnp
from jax.experimental import pallas as pl


def kernel(hidden_states, weight, bias):
    raise NotImplementedError("write your pallas kernel here")


# speedup vs baseline: 2.6227x; 2.6227x over previous
import functools

import jax
import jax.numpy as jnp
from jax import lax
from jax.experimental import pallas as pl
from jax.experimental.pallas import tpu as pltpu
from jax.experimental.pallas import tpu_sc as plsc

_TOP_K = 2
_LANES = 16


def _logits_body(hs_ref, wt_ref, b_ref, out_ref):
    acc = jax.lax.dot_general(
        hs_ref[...], wt_ref[...],
        dimension_numbers=(((1,), (0,)), ((), ())),
        preferred_element_type=jnp.float32,
        precision=jax.lax.Precision.DEFAULT,
    )
    out_ref[...] = acc + b_ref[...]


def _compute_logits(hs, weight_t, bias, block_m):
    tokens, hidden = hs.shape
    num_experts = weight_t.shape[1]
    grid = (tokens // block_m,)
    return pl.pallas_call(
        _logits_body,
        grid=grid,
        in_specs=[
            pl.BlockSpec((block_m, hidden), lambda i: (i, 0)),
            pl.BlockSpec((hidden, num_experts), lambda i: (0, 0)),
            pl.BlockSpec((1, num_experts), lambda i: (0, 0)),
        ],
        out_specs=pl.BlockSpec((block_m, num_experts), lambda i: (i, 0)),
        out_shape=jax.ShapeDtypeStruct((tokens, num_experts), jnp.float32),
    )(hs, weight_t, bias.reshape(1, num_experts))


def _make_router(tokens, num_experts):
    info = plsc.get_sparse_core_info()
    num_workers = info.num_cores * info.num_subcores
    chunk = tokens // num_workers
    groups = chunk // _LANES
    mesh = plsc.VectorSubcoreMesh(core_axis_name="c", subcore_axis_name="s")

    @functools.partial(
        pl.kernel,
        out_type=[
            jax.ShapeDtypeStruct((tokens * num_experts,), jnp.float32),
            jax.ShapeDtypeStruct((tokens * _TOP_K,), jnp.int32),
        ],
        mesh=mesh,
        scratch_types=[
            pltpu.VMEM((chunk * num_experts,), jnp.float32),
            pltpu.VMEM((chunk * num_experts,), jnp.float32),
            pltpu.VMEM((chunk * _TOP_K,), jnp.int32),
        ],
        compiler_params=pltpu.CompilerParams(needs_layout_passes=False),
    )
    def _router(logits_hbm, scores_hbm, idx_hbm, lg_v, sc_v, ix_v):
        wid = lax.axis_index("c") * info.num_subcores + lax.axis_index("s")
        base = wid * chunk
        pltpu.sync_copy(logits_hbm.at[pl.ds(base * num_experts,
                                            chunk * num_experts)], lg_v)
        lanes = lax.iota(jnp.int32, _LANES)

        def body(t, _):
            idx0 = t * (_LANES * num_experts) + lanes * num_experts
            m1 = plsc.load_gather(lg_v, [idx0])
            i1 = jnp.zeros((_LANES,), jnp.int32)
            m2 = jnp.full((_LANES,), -jnp.inf, jnp.float32)
            i2 = jnp.zeros((_LANES,), jnp.int32)
            for e in range(1, num_experts):
                v = plsc.load_gather(lg_v, [idx0 + e])
                ev = jnp.full((_LANES,), e, jnp.int32)
                gt1 = v > m1
                gt2 = v > m2
                i2 = jnp.where(gt1, i1, jnp.where(gt2, ev, i2))
                m2 = jnp.where(gt1, m1, jnp.where(gt2, v, m2))
                i1 = jnp.where(gt1, ev, i1)
                m1 = jnp.where(gt1, v, m1)
            w2 = jnp.exp(m2 - m1)
            p1 = 1.0 / (1.0 + w2)
            p2 = w2 * p1
            zero = jnp.zeros((_LANES,), jnp.float32)
            for e in range(num_experts):
                ev = jnp.full((_LANES,), e, jnp.int32)
                se = jnp.where(i1 == ev, p1, jnp.where(i2 == ev, p2, zero))
                plsc.store_scatter(sc_v, [idx0 + e], se)
            ibase = t * (_LANES * _TOP_K) + lanes * _TOP_K
            plsc.store_scatter(ix_v, [ibase], i1)
            plsc.store_scatter(ix_v, [ibase + 1], i2)
            return ()

        lax.fori_loop(0, groups, body, ())
        pltpu.sync_copy(sc_v, scores_hbm.at[pl.ds(base * num_experts,
                                                  chunk * num_experts)])
        pltpu.sync_copy(ix_v, idx_hbm.at[pl.ds(base * _TOP_K,
                                               chunk * _TOP_K)])

    return _router


def kernel(hidden_states, weight, bias):
    hidden = weight.shape[1]
    num_experts = weight.shape[0]
    hs = hidden_states.reshape(-1, hidden)
    tokens = hs.shape[0]
    logits = _compute_logits(hs, weight.T, bias, block_m=2048)
    router = _make_router(tokens, num_experts)
    scores_flat, idx_flat = router(logits.reshape(-1))
    return (scores_flat.reshape(tokens, num_experts),
            idx_flat.reshape(tokens, _TOP_K))


# speedup vs baseline: 2.6379x; 1.0058x over previous
import functools

import jax
import jax.numpy as jnp
from jax import lax
from jax.experimental import pallas as pl
from jax.experimental.pallas import tpu as pltpu
from jax.experimental.pallas import tpu_sc as plsc

_TOP_K = 2
_LANES = 16


def _logits_body(hs_ref, wt_ref, b_ref, out_ref):
    acc = jax.lax.dot_general(
        hs_ref[...], wt_ref[...],
        dimension_numbers=(((1,), (0,)), ((), ())),
        preferred_element_type=jnp.float32,
        precision=jax.lax.Precision.DEFAULT,
    )
    out_ref[...] = acc + b_ref[...]


def _compute_logits(hs, weight_t, bias, block_m):
    tokens, hidden = hs.shape
    num_experts = weight_t.shape[1]
    grid = (tokens // block_m,)
    return pl.pallas_call(
        _logits_body,
        grid=grid,
        in_specs=[
            pl.BlockSpec((block_m, hidden), lambda i: (i, 0)),
            pl.BlockSpec((hidden, num_experts), lambda i: (0, 0)),
            pl.BlockSpec((1, num_experts), lambda i: (0, 0)),
        ],
        out_specs=pl.BlockSpec((block_m, num_experts), lambda i: (i, 0)),
        out_shape=jax.ShapeDtypeStruct((tokens, num_experts), jnp.float32),
    )(hs, weight_t, bias.reshape(1, num_experts))


def _make_router(tokens, num_experts):
    info = plsc.get_sparse_core_info()
    num_workers = info.num_cores * info.num_subcores
    chunk = tokens // num_workers
    groups = chunk // _LANES
    mesh = plsc.VectorSubcoreMesh(core_axis_name="c", subcore_axis_name="s")

    @functools.partial(
        pl.kernel,
        out_type=[
            jax.ShapeDtypeStruct((tokens * num_experts,), jnp.float32),
            jax.ShapeDtypeStruct((tokens * _TOP_K,), jnp.int32),
        ],
        mesh=mesh,
        scratch_types=[
            pltpu.VMEM((chunk * num_experts,), jnp.float32),
            pltpu.VMEM((chunk * num_experts,), jnp.float32),
            pltpu.VMEM((chunk * _TOP_K,), jnp.int32),
        ],
        compiler_params=pltpu.CompilerParams(needs_layout_passes=False),
    )
    def _router(logits_hbm, scores_hbm, idx_hbm, lg_v, sc_v, ix_v):
        wid = lax.axis_index("c") * info.num_subcores + lax.axis_index("s")
        base = wid * chunk
        pltpu.sync_copy(logits_hbm.at[pl.ds(base * num_experts,
                                            chunk * num_experts)], lg_v)
        lanes = lax.iota(jnp.int32, _LANES)

        def body(t, _):
            idx0 = t * (_LANES * num_experts) + lanes * num_experts
            m1 = plsc.load_gather(lg_v, [idx0])
            i1 = jnp.zeros((_LANES,), jnp.int32)
            m2 = jnp.full((_LANES,), -jnp.inf, jnp.float32)
            i2 = jnp.zeros((_LANES,), jnp.int32)
            for e in range(1, num_experts):
                v = plsc.load_gather(lg_v, [idx0 + e])
                ev = jnp.full((_LANES,), e, jnp.int32)
                gt1 = v > m1
                gt2 = v > m2
                i2 = jnp.where(gt1, i1, jnp.where(gt2, ev, i2))
                m2 = jnp.where(gt1, m1, jnp.where(gt2, v, m2))
                i1 = jnp.where(gt1, ev, i1)
                m1 = jnp.where(gt1, v, m1)
            w2 = jnp.exp(m2 - m1)
            p1 = 1.0 / (1.0 + w2)
            p2 = w2 * p1
            zero = jnp.zeros((_LANES,), jnp.float32)
            for e in range(num_experts):
                ev = jnp.full((_LANES,), e, jnp.int32)
                se = jnp.where(i1 == ev, p1, jnp.where(i2 == ev, p2, zero))
                plsc.store_scatter(sc_v, [idx0 + e], se)
            ibase = t * (_LANES * _TOP_K) + lanes * _TOP_K
            plsc.store_scatter(ix_v, [ibase], i1)
            plsc.store_scatter(ix_v, [ibase + 1], i2)
            return ()

        lax.fori_loop(0, groups, body, ())
        pltpu.sync_copy(sc_v, scores_hbm.at[pl.ds(base * num_experts,
                                                  chunk * num_experts)])
        pltpu.sync_copy(ix_v, idx_hbm.at[pl.ds(base * _TOP_K,
                                               chunk * _TOP_K)])

    return _router


def kernel(hidden_states, weight, bias):
    hidden = weight.shape[1]
    num_experts = weight.shape[0]
    hs = hidden_states.reshape(-1, hidden)
    tokens = hs.shape[0]
    logits = _compute_logits(hs, weight.T, bias, block_m=4096)
    router = _make_router(tokens, num_experts)
    scores_flat, idx_flat = router(logits.reshape(-1))
    return (scores_flat.reshape(tokens, num_experts),
            idx_flat.reshape(tokens, _TOP_K))
